# baseline (speedup 1.0000x reference)
import jax
import jax.numpy as jnp
from jax import lax
from jax.experimental import pallas as pl
from jax.experimental.pallas import tpu as pltpu
from jax.experimental.pallas import tpu_sc as plsc

NC, NS = 2, 16
NW = NC * NS

V = 1000001
D = 64
VBLK = V // 128
TAIL_V0 = VBLK * 128
VPAD = 1000016
TPR = VPAD // 2
EXTRA = VBLK - (VBLK // NW) * NW
BLKS = VBLK // NW


def _transpose_pairs(i16, src, dst):
    fvecs = [f0 * 16 + i16 for f0 in range(4)]

    def rows(r8, carry):
        r0 = r8 * 8
        for dr in range(8):
            r = r0 + dr
            for g in range(8):
                par = g // 4
                col = jnp.full((16,), 0, jnp.int32) + (2 * r + par)
                vals = plsc.load_gather(src, [fvecs[g % 4], col])
                dst[r, pl.ds(g * 16, 16)] = vals
        return carry

    lax.fori_loop(0, 8, rows, 0)


def _tblock_body(wt_hbm, tailp_hbm, tp_hbm, in_v, out_v, tail_v, gsem, wsem):
    wid = lax.axis_index("s") * NC + lax.axis_index("c")
    i16 = lax.iota(jnp.int32, 16)

    def vof(i):
        return i * NW + wid

    def fire(i, sl):
        pltpu.async_copy(
            wt_hbm.at[:, pl.ds(vof(i) * 128, 128)], in_v.at[sl], gsem.at[sl]
        )

    def handle(i, sl):
        pltpu.make_async_copy(
            wt_hbm.at[:, pl.ds(vof(i) * 128, 128)], in_v.at[sl], gsem.at[sl]
        ).wait()

        @pl.when(i >= 2)
        def _():
            pltpu.make_async_copy(
                out_v.at[sl], tp_hbm.at[pl.ds(vof(i - 2) * 64, 64)], wsem.at[sl]
            ).wait()

        _transpose_pairs(i16, in_v.at[sl], out_v.at[sl])
        pltpu.async_copy(
            out_v.at[sl], tp_hbm.at[pl.ds(vof(i) * 64, 64)], wsem.at[sl]
        )

    fire(0, 0)

    def outer(g, carry):
        fire(g + 1, 1)
        handle(g, 0)
        fire(g + 2, 0)
        handle(g + 1, 1)
        return carry

    lax.fori_loop(0, (BLKS - 2) // 2, lambda i, c: outer(2 * i, c), 0)

    fire(BLKS - 1, 1)
    handle(BLKS - 2, 0)
    handle(BLKS - 1, 1)
    pltpu.make_async_copy(
        out_v.at[0], tp_hbm.at[pl.ds(vof(BLKS - 2) * 64, 64)], wsem.at[0]
    ).wait()
    pltpu.make_async_copy(
        out_v.at[1], tp_hbm.at[pl.ds(vof(BLKS - 1) * 64, 64)], wsem.at[1]
    ).wait()

    @pl.when(wid < EXTRA)
    def _():
        v = BLKS * NW + wid
        pltpu.sync_copy(wt_hbm.at[:, pl.ds(v * 128, 128)], in_v.at[0])
        _transpose_pairs(i16, in_v.at[0], out_v.at[0])
        pltpu.sync_copy(out_v.at[0], tp_hbm.at[pl.ds(v * 64, 64)])

    @pl.when(wid == EXTRA)
    def _():
        pltpu.sync_copy(tailp_hbm, tail_v)
        pltpu.sync_copy(
            tail_v, tp_hbm.at[pl.ds(TAIL_V0 // 2, (VPAD - TAIL_V0) // 2)]
        )


def _gather_body(xt_hbm, tl_hbm, out5_hbm, idx_v, rows_v, trow_v, gsem, wsem):
    wid = lax.axis_index("s") * NC + lax.axis_index("c")
    i16 = lax.iota(jnp.int32, 16)
    L = xt_hbm.shape[0]

    pltpu.sync_copy(xt_hbm.at[:, pl.ds(wid * 128, 128)], idx_v)

    def fire(l, sl):
        pltpu.async_copy(tl_hbm.at[idx_v.at[l]], rows_v.at[sl], gsem.at[sl])

    bvecs = [g * 16 + i16 for g in range(8)]

    def transpose(sl):
        def rows8(f8, carry):
            f0 = f8 * 8
            for df in range(8):
                f = f0 + df
                fcol = jnp.full((16,), 0, jnp.int32) + f
                for g in range(8):
                    vals = plsc.load_gather(rows_v.at[sl], [bvecs[g], fcol])
                    trow_v[sl, f, pl.ds(g * 16, 16)] = vals
            return carry

        lax.fori_loop(0, D // 8, rows8, 0)

    def wait_writes(l, sl):
        for dt in range(8):
            pltpu.make_async_copy(
                trow_v.at[sl, pl.ds(dt * 8, 8)],
                out5_hbm.at[l, dt, wid],
                wsem.at[sl],
            ).wait()

    def handle(l, sl):
        pltpu.make_async_copy(
            tl_hbm.at[idx_v.at[l]], rows_v.at[sl], gsem.at[sl]
        ).wait()

        @pl.when(l >= 2)
        def _():
            wait_writes(l - 2, sl)

        transpose(sl)
        for dt in range(8):
            pltpu.async_copy(
                trow_v.at[sl, pl.ds(dt * 8, 8)],
                out5_hbm.at[l, dt, wid],
                wsem.at[sl],
            )

    fire(0, 0)

    def outer(g, carry):
        fire(g + 1, 1)
        handle(g, 0)
        fire(g + 2, 0)
        handle(g + 1, 1)
        return carry

    lax.fori_loop(0, (L - 2) // 2, lambda i, c: outer(2 * i, c), 0)

    fire(L - 1, 1)
    handle(L - 2, 0)
    handle(L - 1, 1)
    wait_writes(L - 2, 0)
    wait_writes(L - 1, 1)


def kernel(x, word_embed):
    B, L = x.shape
    mesh = plsc.VectorSubcoreMesh(
        core_axis_name="c", subcore_axis_name="s", num_cores=NC, num_subcores=NS
    )

    xt = jnp.swapaxes(x.astype(jnp.int32), 0, 1)
    wt = jnp.swapaxes(word_embed, 0, 1)
    tailp = jnp.pad(
        word_embed[TAIL_V0:], ((0, VPAD - V), (0, 0))
    ).reshape((VPAD - TAIL_V0) // 2, 2 * D)

    tp = pl.kernel(
        _tblock_body,
        out_type=jax.ShapeDtypeStruct((TPR, 128), jnp.float32),
        mesh=mesh,
        scratch_types=[
            pltpu.VMEM((2, 64, 128), jnp.float32),
            pltpu.VMEM((2, 64, 128), jnp.float32),
            pltpu.VMEM(((VPAD - TAIL_V0) // 2, 128), jnp.float32),
            pltpu.SemaphoreType.DMA((2,)),
            pltpu.SemaphoreType.DMA((2,)),
        ],
        compiler_params=pltpu.CompilerParams(
            use_tc_tiling_on_sc=True, needs_layout_passes=False
        ),
    )(wt, tailp)

    tl = tp.reshape(VPAD, D)

    out5 = pl.kernel(
        _gather_body,
        out_type=jax.ShapeDtypeStruct((L, D // 8, B // 128, 8, 128), jnp.float32),
        mesh=mesh,
        scratch_types=[
            pltpu.VMEM((L, 128), jnp.int32),
            pltpu.VMEM((2, 128, D), jnp.float32),
            pltpu.VMEM((2, D, 128), jnp.float32),
            pltpu.SemaphoreType.DMA((2,)),
            pltpu.SemaphoreType.DMA((2,)),
        ],
        compiler_params=pltpu.CompilerParams(
            use_tc_tiling_on_sc=False, needs_layout_passes=False
        ),
    )(xt, tl)

    return out5.transpose(2, 4, 0, 1, 3).reshape(B, L, D)


# speedup vs baseline: 2.4732x; 2.4732x over previous
import jax
import jax.numpy as jnp
from jax import lax
from jax.experimental import pallas as pl
from jax.experimental.pallas import tpu as pltpu
from jax.experimental.pallas import tpu_sc as plsc

NC, NS = 2, 16
NW = NC * NS

V = 1000001
D = 64
VBLK = V // 128
TAIL_V0 = VBLK * 128
VPAD = 1000016
TPR = VPAD // 2
EXTRA = VBLK - (VBLK // NW) * NW
BLKS = VBLK // NW
TAILN = (VPAD - TAIL_V0) * D


def _diag_transpose(i16, src, dst_flat, sh, sw):
    nr, nc = sh // 16, sw // 16

    def tile(t, carry):
        r0 = lax.rem(t, nr) * 16
        c0 = lax.div(t, nr) * 16
        rvec = r0 + i16

        def diag(d, c2):
            cvec = c0 + ((i16 + d) & 15)
            vals = plsc.load_gather(src, [rvec, cvec])
            plsc.store_scatter(dst_flat, [cvec * sh + rvec], vals)
            return c2

        lax.fori_loop(0, 16, diag, 0)
        return carry

    lax.fori_loop(0, nr * nc, tile, 0)


def _tblock_body(wt_hbm, tailp_hbm, tp_hbm, in0, in1, out0, out1, tail_v,
                 gsem, wsem):
    wid = lax.axis_index("s") * NC + lax.axis_index("c")
    i16 = lax.iota(jnp.int32, 16)
    ins, outs = [in0, in1], [out0, out1]

    def vof(i):
        return i * NW + wid

    def fire(i, sl):
        pltpu.async_copy(
            wt_hbm.at[:, pl.ds(vof(i) * 128, 128)], ins[sl], gsem.at[sl]
        )

    def handle(i, sl):
        pltpu.make_async_copy(
            wt_hbm.at[:, pl.ds(vof(i) * 128, 128)], ins[sl], gsem.at[sl]
        ).wait()

        @pl.when(i >= 2)
        def _():
            pltpu.make_async_copy(
                outs[sl], tp_hbm.at[pl.ds(vof(i - 2) * 8192, 8192)],
                wsem.at[sl],
            ).wait()

        _diag_transpose(i16, ins[sl], outs[sl], 64, 128)
        pltpu.async_copy(
            outs[sl], tp_hbm.at[pl.ds(vof(i) * 8192, 8192)], wsem.at[sl]
        )

    fire(0, 0)

    def outer(g, carry):
        fire(g + 1, 1)
        handle(g, 0)
        fire(g + 2, 0)
        handle(g + 1, 1)
        return carry

    lax.fori_loop(0, (BLKS - 2) // 2, lambda i, c: outer(2 * i, c), 0)

    fire(BLKS - 1, 1)
    handle(BLKS - 2, 0)
    handle(BLKS - 1, 1)
    pltpu.make_async_copy(
        outs[0], tp_hbm.at[pl.ds(vof(BLKS - 2) * 8192, 8192)], wsem.at[0]
    ).wait()
    pltpu.make_async_copy(
        outs[1], tp_hbm.at[pl.ds(vof(BLKS - 1) * 8192, 8192)], wsem.at[1]
    ).wait()

    @pl.when(wid < EXTRA)
    def _():
        v = BLKS * NW + wid
        pltpu.sync_copy(wt_hbm.at[:, pl.ds(v * 128, 128)], ins[0])
        _diag_transpose(i16, ins[0], outs[0], 64, 128)
        pltpu.sync_copy(outs[0], tp_hbm.at[pl.ds(v * 8192, 8192)])

    @pl.when(wid == EXTRA)
    def _():
        pltpu.sync_copy(tailp_hbm, tail_v)
        pltpu.sync_copy(tail_v, tp_hbm.at[pl.ds(TAIL_V0 * D, TAILN)])


def _gather_body(xt_hbm, tl_hbm, out_hbm, idx_v, rows0, rows1, trow0, trow1,
                 gsem, wsem):
    wid = lax.axis_index("s") * NC + lax.axis_index("c")
    i16 = lax.iota(jnp.int32, 16)
    L = xt_hbm.shape[0]
    rows, trows = [rows0, rows1], [trow0, trow1]

    pltpu.sync_copy(xt_hbm.at[:, pl.ds(wid * 128, 128)], idx_v)

    def fire(l, sl):
        pltpu.async_copy(tl_hbm.at[idx_v.at[l]], rows[sl], gsem.at[sl])

    def obase(l, dt):
        return ((l * 8 + dt) * NW + wid) * 1024

    def wait_writes(l, sl):
        for dt in range(8):
            pltpu.make_async_copy(
                trows[sl].at[pl.ds(dt * 1024, 1024)],
                out_hbm.at[pl.ds(obase(l, dt), 1024)],
                wsem.at[sl],
            ).wait()

    def handle(l, sl):
        pltpu.make_async_copy(
            tl_hbm.at[idx_v.at[l]], rows[sl], gsem.at[sl]
        ).wait()

        @pl.when(l >= 2)
        def _():
            wait_writes(l - 2, sl)

        _diag_transpose(i16, rows[sl], trows[sl], 128, 64)
        for dt in range(8):
            pltpu.async_copy(
                trows[sl].at[pl.ds(dt * 1024, 1024)],
                out_hbm.at[pl.ds(obase(l, dt), 1024)],
                wsem.at[sl],
            )

    fire(0, 0)

    def outer(g, carry):
        fire(g + 1, 1)
        handle(g, 0)
        fire(g + 2, 0)
        handle(g + 1, 1)
        return carry

    lax.fori_loop(0, (L - 2) // 2, lambda i, c: outer(2 * i, c), 0)

    fire(L - 1, 1)
    handle(L - 2, 0)
    handle(L - 1, 1)
    wait_writes(L - 2, 0)
    wait_writes(L - 1, 1)


def kernel(x, word_embed):
    B, L = x.shape
    mesh = plsc.VectorSubcoreMesh(
        core_axis_name="c", subcore_axis_name="s", num_cores=NC, num_subcores=NS
    )

    xt = jnp.swapaxes(x.astype(jnp.int32), 0, 1)
    wt = jnp.swapaxes(word_embed, 0, 1)
    tailp = jnp.pad(
        word_embed[TAIL_V0:], ((0, VPAD - V), (0, 0))
    ).reshape(TAILN)

    tp = pl.kernel(
        _tblock_body,
        out_type=jax.ShapeDtypeStruct((TPR * 128,), jnp.float32),
        mesh=mesh,
        scratch_types=[
            pltpu.VMEM((64, 128), jnp.float32),
            pltpu.VMEM((64, 128), jnp.float32),
            pltpu.VMEM((8192,), jnp.float32),
            pltpu.VMEM((8192,), jnp.float32),
            pltpu.VMEM((TAILN,), jnp.float32),
            pltpu.SemaphoreType.DMA((2,)),
            pltpu.SemaphoreType.DMA((2,)),
        ],
        compiler_params=pltpu.CompilerParams(
            use_tc_tiling_on_sc=True, needs_layout_passes=False
        ),
    )(wt, tailp)

    tl = tp.reshape(VPAD, D)

    out_flat = pl.kernel(
        _gather_body,
        out_type=jax.ShapeDtypeStruct((B * L * D,), jnp.float32),
        mesh=mesh,
        scratch_types=[
            pltpu.VMEM((L, 128), jnp.int32),
            pltpu.VMEM((128, D), jnp.float32),
            pltpu.VMEM((128, D), jnp.float32),
            pltpu.VMEM((8192,), jnp.float32),
            pltpu.VMEM((8192,), jnp.float32),
            pltpu.SemaphoreType.DMA((2,)),
            pltpu.SemaphoreType.DMA((2,)),
        ],
        compiler_params=pltpu.CompilerParams(
            use_tc_tiling_on_sc=False, needs_layout_passes=False
        ),
    )(xt, tl)

    out5 = out_flat.reshape(L, D // 8, B // 128, 8, 128)
    return out5.transpose(2, 4, 0, 1, 3).reshape(B, L, D)


# speedup vs baseline: 2.8064x; 1.1347x over previous
import jax
import jax.numpy as jnp
from jax import lax
from jax.experimental import pallas as pl
from jax.experimental.pallas import tpu as pltpu
from jax.experimental.pallas import tpu_sc as plsc

NC, NS = 2, 16
NW = NC * NS

V = 1000001
D = 64
VBLK = V // 128
TAIL_V0 = VBLK * 128
VPAD = 1000016
TPR = VPAD // 2
EXTRA = VBLK - (VBLK // NW) * NW
BLKS = VBLK // NW
TAILN = (VPAD - TAIL_V0) * D


def _diag_transpose(i16, src, dst_flat, sh, sw):
    nr, nc = sh // 16, sw // 16

    def tile(t, carry):
        r0 = lax.rem(t, nr) * 16
        c0 = lax.div(t, nr) * 16
        rvec = r0 + i16

        def diag(d4, c2):
            for u in range(4):
                cvec = c0 + ((i16 + (d4 * 4 + u)) & 15)
                vals = plsc.load_gather(src, [rvec, cvec])
                plsc.store_scatter(dst_flat, [cvec * sh + rvec], vals)
            return c2

        lax.fori_loop(0, 4, diag, 0)
        return carry

    lax.fori_loop(0, nr * nc, tile, 0)


def _tblock_body(wt_hbm, tailp_hbm, tp_hbm, in0, in1, out0, out1, tail_v,
                 gsem, wsem):
    wid = lax.axis_index("s") * NC + lax.axis_index("c")
    i16 = lax.iota(jnp.int32, 16)
    ins, outs = [in0, in1], [out0, out1]

    def vof(i):
        return i * NW + wid

    def fire(i, sl):
        pltpu.async_copy(
            wt_hbm.at[:, pl.ds(vof(i) * 128, 128)], ins[sl], gsem.at[sl]
        )

    def handle(i, sl):
        pltpu.make_async_copy(
            wt_hbm.at[:, pl.ds(vof(i) * 128, 128)], ins[sl], gsem.at[sl]
        ).wait()

        @pl.when(i >= 2)
        def _():
            pltpu.make_async_copy(
                outs[sl], tp_hbm.at[pl.ds(vof(i - 2) * 8192, 8192)],
                wsem.at[sl],
            ).wait()

        _diag_transpose(i16, ins[sl], outs[sl], 64, 128)
        pltpu.async_copy(
            outs[sl], tp_hbm.at[pl.ds(vof(i) * 8192, 8192)], wsem.at[sl]
        )

    fire(0, 0)

    def outer(g, carry):
        fire(g + 1, 1)
        handle(g, 0)
        fire(g + 2, 0)
        handle(g + 1, 1)
        return carry

    lax.fori_loop(0, (BLKS - 2) // 2, lambda i, c: outer(2 * i, c), 0)

    fire(BLKS - 1, 1)
    handle(BLKS - 2, 0)
    handle(BLKS - 1, 1)
    pltpu.make_async_copy(
        outs[0], tp_hbm.at[pl.ds(vof(BLKS - 2) * 8192, 8192)], wsem.at[0]
    ).wait()
    pltpu.make_async_copy(
        outs[1], tp_hbm.at[pl.ds(vof(BLKS - 1) * 8192, 8192)], wsem.at[1]
    ).wait()

    @pl.when(wid < EXTRA)
    def _():
        v = BLKS * NW + wid
        pltpu.sync_copy(wt_hbm.at[:, pl.ds(v * 128, 128)], ins[0])
        _diag_transpose(i16, ins[0], outs[0], 64, 128)
        pltpu.sync_copy(outs[0], tp_hbm.at[pl.ds(v * 8192, 8192)])

    @pl.when(wid == EXTRA)
    def _():
        pltpu.sync_copy(tailp_hbm, tail_v)
        pltpu.sync_copy(tail_v, tp_hbm.at[pl.ds(TAIL_V0 * D, TAILN)])


def _gather_body(xt_hbm, tl_hbm, out_hbm, idx_v, rows0, rows1, trow0, trow1,
                 gsem, wsem):
    wid = lax.axis_index("s") * NC + lax.axis_index("c")
    i16 = lax.iota(jnp.int32, 16)
    L = xt_hbm.shape[0]
    rows, trows = [rows0, rows1], [trow0, trow1]

    pltpu.sync_copy(xt_hbm.at[:, pl.ds(wid * 128, 128)], idx_v)

    def fire(l, sl):
        pltpu.async_copy(tl_hbm.at[idx_v.at[l]], rows[sl], gsem.at[sl])

    def obase(l, dt):
        return ((l * 8 + dt) * NW + wid) * 1024

    def wait_writes(l, sl):
        for dt in range(8):
            pltpu.make_async_copy(
                trows[sl].at[pl.ds(dt * 1024, 1024)],
                out_hbm.at[pl.ds(obase(l, dt), 1024)],
                wsem.at[sl],
            ).wait()

    def handle(l, sl):
        pltpu.make_async_copy(
            tl_hbm.at[idx_v.at[l]], rows[sl], gsem.at[sl]
        ).wait()

        @pl.when(l >= 2)
        def _():
            wait_writes(l - 2, sl)

        _diag_transpose(i16, rows[sl], trows[sl], 128, 64)
        for dt in range(8):
            pltpu.async_copy(
                trows[sl].at[pl.ds(dt * 1024, 1024)],
                out_hbm.at[pl.ds(obase(l, dt), 1024)],
                wsem.at[sl],
            )

    fire(0, 0)

    def outer(g, carry):
        fire(g + 1, 1)
        handle(g, 0)
        fire(g + 2, 0)
        handle(g + 1, 1)
        return carry

    lax.fori_loop(0, (L - 2) // 2, lambda i, c: outer(2 * i, c), 0)

    fire(L - 1, 1)
    handle(L - 2, 0)
    handle(L - 1, 1)
    wait_writes(L - 2, 0)
    wait_writes(L - 1, 1)


def kernel(x, word_embed):
    B, L = x.shape
    mesh = plsc.VectorSubcoreMesh(
        core_axis_name="c", subcore_axis_name="s", num_cores=NC, num_subcores=NS
    )

    xt = jnp.swapaxes(x.astype(jnp.int32), 0, 1)
    wt = jnp.swapaxes(word_embed, 0, 1)
    tailp = jnp.pad(
        word_embed[TAIL_V0:], ((0, VPAD - V), (0, 0))
    ).reshape(TAILN)

    tp = pl.kernel(
        _tblock_body,
        out_type=jax.ShapeDtypeStruct((TPR * 128,), jnp.float32),
        mesh=mesh,
        scratch_types=[
            pltpu.VMEM((64, 128), jnp.float32),
            pltpu.VMEM((64, 128), jnp.float32),
            pltpu.VMEM((8192,), jnp.float32),
            pltpu.VMEM((8192,), jnp.float32),
            pltpu.VMEM((TAILN,), jnp.float32),
            pltpu.SemaphoreType.DMA((2,)),
            pltpu.SemaphoreType.DMA((2,)),
        ],
        compiler_params=pltpu.CompilerParams(
            use_tc_tiling_on_sc=True, needs_layout_passes=False
        ),
    )(wt, tailp)

    tl = tp.reshape(VPAD, D)

    out_flat = pl.kernel(
        _gather_body,
        out_type=jax.ShapeDtypeStruct((B * L * D,), jnp.float32),
        mesh=mesh,
        scratch_types=[
            pltpu.VMEM((L, 128), jnp.int32),
            pltpu.VMEM((128, D), jnp.float32),
            pltpu.VMEM((128, D), jnp.float32),
            pltpu.VMEM((8192,), jnp.float32),
            pltpu.VMEM((8192,), jnp.float32),
            pltpu.SemaphoreType.DMA((2,)),
            pltpu.SemaphoreType.DMA((2,)),
        ],
        compiler_params=pltpu.CompilerParams(
            use_tc_tiling_on_sc=False, needs_layout_passes=False
        ),
    )(xt, tl)

    out5 = out_flat.reshape(L, D // 8, B // 128, 8, 128)
    return out5.transpose(2, 4, 0, 1, 3).reshape(B, L, D)
